# Initial kernel scaffold; baseline (speedup 1.0000x reference)
import jax
import jax.numpy as jnp
from jax.experimental import pallas as pl


def kernel(x, edge_index, W1l, b1, W1r, W2l, b2, W2r, W3l, b3, W3r, Wo, bo):
    raise NotImplementedError("write your pallas kernel here")


# speedup vs baseline: 15.9565x; 15.9565x over previous
import functools

import jax
import jax.numpy as jnp
from jax import lax
from jax.experimental import pallas as pl
from jax.experimental.pallas import tpu as pltpu
from jax.experimental.pallas import tpu_sc as plsc

_N = 10000
_E = 320000
_CH = 32
_NPAD = 10240
_CHUNK = 80
_NCHUNKS = _E // _CHUNK
_NC, _NS = 2, 16
_NW = _NC * _NS
_CPT = _NCHUNKS // _NW
_ROWS_PER_TILE = _NPAD // _NS


def _sc_aggregate(with_deg):
    out_type = [jax.ShapeDtypeStruct((_NC, _NPAD, _CH), jnp.float32)]
    scratch = [
        pltpu.VMEM((_CPT, _CHUNK), jnp.int32),
        pltpu.VMEM((_CPT, _CHUNK), jnp.int32),
        pltpu.VMEM((_CHUNK, _CH), jnp.float32),
        pltpu.VMEM((_CHUNK, _CH), jnp.float32),
        pltpu.VMEM_SHARED((_NPAD, _CH), jnp.float32),
        pltpu.SemaphoreType.DMA,
        pltpu.SemaphoreType.DMA,
    ]
    if with_deg:
        out_type.append(jax.ShapeDtypeStruct((_NC, _NPAD, _CH), jnp.float32))
        scratch += [
            pltpu.VMEM((_CHUNK, _CH), jnp.float32),
            pltpu.VMEM_SHARED((_NPAD, _CH), jnp.float32),
        ]

    mesh = plsc.VectorSubcoreMesh(core_axis_name="c", subcore_axis_name="s")

    def body(xs, src3d, dst3d, zeros, *rest):
        if with_deg:
            (ones_hbm, agg_out, deg_out,
             src_v, dst_v, rows_a, rows_b, acc, sem_a, sem_b,
             ones_v, dacc) = rest
        else:
            (agg_out,
             src_v, dst_v, rows_a, rows_b, acc, sem_a, sem_b) = rest

        c = lax.axis_index("c")
        s = lax.axis_index("s")
        w = s * _NC + c

        r0 = s * _ROWS_PER_TILE
        pltpu.sync_copy(zeros.at[pl.ds(r0, _ROWS_PER_TILE)],
                        acc.at[pl.ds(r0, _ROWS_PER_TILE)])
        if with_deg:
            pltpu.sync_copy(zeros.at[pl.ds(r0, _ROWS_PER_TILE)],
                            dacc.at[pl.ds(r0, _ROWS_PER_TILE)])
            pltpu.sync_copy(ones_hbm, ones_v)

        pltpu.sync_copy(src3d.at[w], src_v)
        pltpu.sync_copy(dst3d.at[w], dst_v)

        plsc.subcore_barrier()

        pltpu.async_copy(xs.at[src_v.at[0]], rows_a, sem_a)

        def step(i, carry):
            j = 2 * i
            pltpu.async_copy(xs.at[src_v.at[j + 1]], rows_b, sem_b)
            pltpu.make_async_copy(xs.at[src_v.at[j]], rows_a, sem_a).wait()
            pltpu.sync_copy(rows_a, acc.at[dst_v.at[j]], add=True)
            if with_deg:
                pltpu.sync_copy(ones_v, dacc.at[dst_v.at[j]], add=True)
            pltpu.async_copy(xs.at[src_v.at[j + 2]], rows_a, sem_a)
            pltpu.make_async_copy(xs.at[src_v.at[j + 1]], rows_b, sem_b).wait()
            pltpu.sync_copy(rows_b, acc.at[dst_v.at[j + 1]], add=True)
            if with_deg:
                pltpu.sync_copy(ones_v, dacc.at[dst_v.at[j + 1]], add=True)
            return carry

        lax.fori_loop(0, (_CPT - 1) // 2, step, 0)

        jlast = _CPT - 1
        pltpu.make_async_copy(xs.at[src_v.at[jlast]], rows_a, sem_a).wait()
        pltpu.sync_copy(rows_a, acc.at[dst_v.at[jlast]], add=True)
        if with_deg:
            pltpu.sync_copy(ones_v, dacc.at[dst_v.at[jlast]], add=True)

        plsc.subcore_barrier()

        pltpu.sync_copy(acc.at[pl.ds(r0, _ROWS_PER_TILE)],
                        agg_out.at[c, pl.ds(r0, _ROWS_PER_TILE)])
        if with_deg:
            pltpu.sync_copy(dacc.at[pl.ds(r0, _ROWS_PER_TILE)],
                            deg_out.at[c, pl.ds(r0, _ROWS_PER_TILE)])

    return pl.kernel(body, out_type=out_type, mesh=mesh, scratch_types=scratch,
                     compiler_params=pltpu.CompilerParams(
                         use_tc_tiling_on_sc=False))


def _tc_proj(x_ref, wl_ref, wr_ref, xs_ref, hr_ref):
    x = x_ref[...]
    xs_ref[...] = jnp.dot(x, wl_ref[...], preferred_element_type=jnp.float32)
    hr_ref[...] = jnp.dot(x, wr_ref[...], preferred_element_type=jnp.float32)


def _tc_combine_proj(agg_ref, deg_ref, hr_ref, b_ref, wl_ref, wr_ref,
                     xs_ref, hrn_ref):
    agg = agg_ref[0, :_N, :] + agg_ref[1, :_N, :]
    deg = deg_ref[0, :_N, :] + deg_ref[1, :_N, :]
    inv = 1.0 / jnp.maximum(deg, 1.0)
    h = jnp.maximum(agg * inv + b_ref[...] + hr_ref[...], 0.0)
    xs_ref[...] = jnp.dot(h, wl_ref[...], preferred_element_type=jnp.float32)
    hrn_ref[...] = jnp.dot(h, wr_ref[...], preferred_element_type=jnp.float32)


def _tc_combine_final(agg_ref, deg_ref, hr_ref, b_ref, wo_ref, bo_ref,
                      out_ref):
    agg = agg_ref[0, :_N, :] + agg_ref[1, :_N, :]
    deg = deg_ref[0, :_N, :] + deg_ref[1, :_N, :]
    inv = 1.0 / jnp.maximum(deg, 1.0)
    h = jnp.maximum(agg * inv + b_ref[...] + hr_ref[...], 0.0)
    out_ref[...] = (jnp.dot(h, wo_ref[...], preferred_element_type=jnp.float32)
                    + bo_ref[...])


_f32 = jnp.float32


@jax.jit
def kernel(x, edge_index, W1l, b1, W1r, W2l, b2, W2r, W3l, b3, W3r, Wo, bo):
    src2d = edge_index[0].reshape(_NW, _CPT, _CHUNK)
    dst2d = edge_index[1].reshape(_NW, _CPT, _CHUNK)
    zeros = jnp.zeros((_NPAD, _CH), _f32)
    ones = jnp.ones((_CHUNK, _CH), _f32)

    two_proj = pl.pallas_call(
        _tc_proj,
        out_shape=[jax.ShapeDtypeStruct((_N, _CH), _f32),
                   jax.ShapeDtypeStruct((_N, _CH), _f32)],
    )
    combine_proj = pl.pallas_call(
        _tc_combine_proj,
        out_shape=[jax.ShapeDtypeStruct((_N, _CH), _f32),
                   jax.ShapeDtypeStruct((_N, _CH), _f32)],
    )
    combine_final = pl.pallas_call(
        _tc_combine_final,
        out_shape=jax.ShapeDtypeStruct((_N, 128), _f32),
    )

    agg_deg = _sc_aggregate(True)
    agg_only = _sc_aggregate(False)

    xs1, hr1 = two_proj(x, W1l.T, W1r.T)
    agg1, deg = agg_deg(xs1, src2d, dst2d, zeros, ones)
    xs2, hr2 = combine_proj(agg1, deg, hr1, b1.reshape(1, _CH), W2l.T, W2r.T)
    (agg2,) = agg_only(xs2, src2d, dst2d, zeros)
    xs3, hr3 = combine_proj(agg2, deg, hr2, b2.reshape(1, _CH), W3l.T, W3r.T)
    (agg3,) = agg_only(xs3, src2d, dst2d, zeros)
    out = combine_final(agg3, deg, hr3, b3.reshape(1, _CH), Wo.T,
                        bo.reshape(1, 128))
    return out
